# Initial kernel scaffold; baseline (speedup 1.0000x reference)
import functools

import jax
import jax.numpy as jnp
from jax.experimental import pallas as pl
from jax.experimental.pallas import tpu as pltpu


def _mixed_kernel(xp_ref, w0_ref, b0_ref, w12_ref, b12_ref,
                  w1_ref, b1_ref, w2_ref, b2_ref, w3_ref, b3_ref,
                  o_ref, *, D, H, W, C1, C2):
    DP, HP, WP = D + 2, H + 2, W + 2
    M = D * H * W
    MP = DP * HP * WP
    xp = xp_ref[...]
    C = xp.shape[-1]

    dd = jax.lax.broadcasted_iota(jnp.int32, (DP, HP, WP, 1), 0)
    hh = jax.lax.broadcasted_iota(jnp.int32, (DP, HP, WP, 1), 1)
    ww = jax.lax.broadcasted_iota(jnp.int32, (DP, HP, WP, 1), 2)
    interior = ((dd >= 1) & (dd <= D) & (hh >= 1) & (hh <= H)
                & (ww >= 1) & (ww <= W))

    hid = jnp.dot(xp.reshape(MP, C), w12_ref[...],
                  preferred_element_type=jnp.float32)
    hid = jnp.maximum(hid + b12_ref[...], 0.0)
    hid = jnp.where(interior.reshape(MP, 1), hid, 0.0).astype(jnp.bfloat16)
    hid = hid.reshape(DP, HP, WP, C1 + C2)
    h1 = hid[..., :C1]
    h2 = hid[..., C1:]

    xin = xp[1:1 + D, 1:1 + H, 1:1 + W, :].reshape(M, C)
    y0 = jnp.maximum(
        jnp.dot(xin, w0_ref[...], preferred_element_type=jnp.float32)
        + b0_ref[...], 0.0)

    acc1 = jnp.zeros((M, w1_ref.shape[-1]), jnp.float32)
    for kd in range(3):
        taps = [h1[kd:kd + D, kh:kh + H, kw:kw + W, :]
                for kh in range(3) for kw in range(3)]
        wide = jnp.concatenate(taps, axis=-1).reshape(M, 9 * C1)
        acc1 = acc1 + jnp.dot(wide, w1_ref[kd],
                              preferred_element_type=jnp.float32)
    y1 = jnp.maximum(acc1 + b1_ref[...], 0.0)

    taps2 = [h2[kd:kd + D, kh:kh + H, kw:kw + W, :]
             for kd in range(3) for kh in range(3) for kw in range(3)]
    wide2 = jnp.concatenate(taps2, axis=-1).reshape(M, 27 * C2)
    y2 = jnp.maximum(
        jnp.dot(wide2, w2_ref[...], preferred_element_type=jnp.float32)
        + b2_ref[...], 0.0)

    xm = jnp.where(interior, xp, jnp.asarray(-jnp.inf, xp.dtype))
    pooled = None
    for kd in range(3):
        for kh in range(3):
            for kw in range(3):
                tap = xm[kd:kd + D, kh:kh + H, kw:kw + W, :]
                pooled = tap if pooled is None else jnp.maximum(pooled, tap)
    y3 = jnp.maximum(
        jnp.dot(pooled.reshape(M, C), w3_ref[...],
                preferred_element_type=jnp.float32) + b3_ref[...], 0.0)

    out = jnp.concatenate([y0, y1, y2, y3], axis=-1)
    o_ref[...] = out.reshape(D, H, W, out.shape[-1]).astype(o_ref.dtype)


def kernel(x,
           b0_w, b0_s, b0_b,
           b1a_w, b1a_s, b1a_b,
           b1b_w, b1b_s, b1b_b,
           b2a_w, b2a_s, b2a_b,
           b2b_w, b2b_s, b2b_b,
           b3_w, b3_s, b3_b):
    n, c, d, h, w = x.shape
    bf = jnp.bfloat16
    xt = jnp.transpose(x, (0, 2, 3, 4, 1)).astype(bf)
    xp = jnp.pad(xt, ((0, 0), (1, 1), (1, 1), (1, 1), (0, 0)))

    c0 = b0_w.shape[1]
    c1 = b1a_w.shape[1]
    c2 = b2a_w.shape[1]
    c1b = b1b_w.shape[-1]
    c2b = b2b_w.shape[-1]
    c3 = b3_w.shape[1]
    couts = c0 + c1b + c2b + c3

    w0f = (b0_w * b0_s[None, :]).astype(bf)
    w12 = jnp.concatenate([b1a_w * b1a_s[None, :],
                           b2a_w * b2a_s[None, :]], axis=1).astype(bf)
    b12 = jnp.concatenate([b1a_b, b2a_b]).reshape(1, c1 + c2)
    w1f = (b1b_w * b1b_s).reshape(3, 9 * c1, c1b).astype(bf)
    w2f = (b2b_w * b2b_s).reshape(27 * c2, c2b).astype(bf)
    w3f = (b3_w * b3_s[None, :]).astype(bf)

    dp, hp, wp = d + 2, h + 2, w + 2
    out = pl.pallas_call(
        functools.partial(_mixed_kernel, D=d, H=h, W=w, C1=c1, C2=c2),
        out_shape=jax.ShapeDtypeStruct((n, d, h, w, couts), jnp.float32),
        grid_spec=pltpu.PrefetchScalarGridSpec(
            num_scalar_prefetch=0,
            grid=(n,),
            in_specs=[
                pl.BlockSpec((pl.Squeezed(), dp, hp, wp, c),
                             lambda ni: (ni, 0, 0, 0, 0)),
                pl.BlockSpec((c, c0), lambda ni: (0, 0)),
                pl.BlockSpec((1, c0), lambda ni: (0, 0)),
                pl.BlockSpec((c, c1 + c2), lambda ni: (0, 0)),
                pl.BlockSpec((1, c1 + c2), lambda ni: (0, 0)),
                pl.BlockSpec((3, 9 * c1, c1b), lambda ni: (0, 0, 0)),
                pl.BlockSpec((1, c1b), lambda ni: (0, 0)),
                pl.BlockSpec((27 * c2, c2b), lambda ni: (0, 0)),
                pl.BlockSpec((1, c2b), lambda ni: (0, 0)),
                pl.BlockSpec((c, c3), lambda ni: (0, 0)),
                pl.BlockSpec((1, c3), lambda ni: (0, 0)),
            ],
            out_specs=pl.BlockSpec((pl.Squeezed(), d, h, w, couts),
                                   lambda ni: (ni, 0, 0, 0, 0)),
        ),
        compiler_params=pltpu.CompilerParams(
            dimension_semantics=("parallel",),
            vmem_limit_bytes=100 * 1024 * 1024,
        ),
    )(xp, w0f, b0_b.reshape(1, c0), w12, b12,
      w1f, b1b_b.reshape(1, c1b), w2f, b2b_b.reshape(1, c2b),
      w3f, b3_b.reshape(1, c3))
    return jnp.transpose(out, (0, 4, 1, 2, 3))


# speedup vs baseline: 2.7914x; 2.7914x over previous
import functools

import jax
import jax.numpy as jnp
from jax.experimental import pallas as pl
from jax.experimental.pallas import tpu as pltpu


def _round_up(x, m):
    return (x + m - 1) // m * m


def _mixed_kernel(xp_ref, w0_ref, b0_ref, w12_ref, b12_ref,
                  w1_ref, b1_ref, w2_ref, b2_ref, w3_ref, b3_ref, sm_ref,
                  o_ref, *, DB, D, H, W, C1, C2, SP, PADS):
    d0 = pl.multiple_of(pl.program_id(1) * DB, DB)
    DS = DB + 2
    WP = W + 2
    S = (H + 2) * WP
    SH = H * WP
    M = DB * SH
    xs = xp_ref[pl.ds(d0, DS)]
    C = xs.shape[-1]

    dd = d0 + jax.lax.broadcasted_iota(jnp.int32, (DS, 1, 1), 0)
    dvalid = (dd >= 1) & (dd <= D)
    svalid = (sm_ref[...] != 0)[None, :, :]
    interior = dvalid & svalid

    hid = jnp.dot(xs.reshape(DS * SP, C), w12_ref[...],
                  preferred_element_type=jnp.float32)
    hid = jnp.maximum(hid + b12_ref[...], 0.0)
    hid = jnp.where(interior.reshape(DS * SP, 1), hid, 0.0)
    hs = hid.astype(jnp.bfloat16).reshape(DS, SP, C1 + C2)

    r0 = PADS + WP
    xin = xs[1:1 + DB, r0:r0 + SH, :].reshape(M, C)
    y0 = jnp.maximum(
        jnp.dot(xin, w0_ref[...], preferred_element_type=jnp.float32)
        + b0_ref[...], 0.0)

    acc1 = jnp.zeros((M, w1_ref.shape[-1]), jnp.float32)
    for kd in range(3):
        taps = [hs[kd:kd + DB,
                   PADS + kh * WP + kw - 1:PADS + kh * WP + kw - 1 + SH,
                   :C1]
                for kh in range(3) for kw in range(3)]
        wide = jnp.concatenate(taps, axis=-1).reshape(M, 9 * C1)
        acc1 = acc1 + jnp.dot(wide, w1_ref[kd],
                              preferred_element_type=jnp.float32)
    y1 = jnp.maximum(acc1 + b1_ref[...], 0.0)

    taps2 = [hs[kd:kd + DB,
                PADS + kh * WP + kw - 1:PADS + kh * WP + kw - 1 + SH,
                C1:]
             for kd in range(3) for kh in range(3) for kw in range(3)]
    wide2 = jnp.concatenate(taps2, axis=-1).reshape(M, 27 * C2)
    y2 = jnp.maximum(
        jnp.dot(wide2, w2_ref[...], preferred_element_type=jnp.float32)
        + b2_ref[...], 0.0)

    xm = jnp.where(interior, xs, jnp.asarray(-jnp.inf, xs.dtype))
    mw = jnp.maximum(jnp.maximum(xm[:, 0:SP - 2, :], xm[:, 1:SP - 1, :]),
                     xm[:, 2:SP, :])
    mh = jnp.maximum(jnp.maximum(mw[:, 0:S, :], mw[:, WP:WP + S, :]),
                     mw[:, 2 * WP:2 * WP + S, :])
    mhc = mh[:, WP:WP + SH, :]
    pooled = jnp.maximum(jnp.maximum(mhc[0:DB], mhc[1:1 + DB]), mhc[2:2 + DB])
    y3 = jnp.maximum(
        jnp.dot(pooled.reshape(M, C), w3_ref[...],
                preferred_element_type=jnp.float32) + b3_ref[...], 0.0)

    out = jnp.concatenate([y0, y1, y2, y3], axis=-1)
    out = out.reshape(DB, H, WP, out.shape[-1])[:, :, 1:1 + W, :]
    o_ref[...] = out.astype(o_ref.dtype)


def kernel(x,
           b0_w, b0_s, b0_b,
           b1a_w, b1a_s, b1a_b,
           b1b_w, b1b_s, b1b_b,
           b2a_w, b2a_s, b2a_b,
           b2b_w, b2b_s, b2b_b,
           b3_w, b3_s, b3_b):
    n, c, d, h, w = x.shape
    bf = jnp.bfloat16
    dp, hp, wp = d + 2, h + 2, w + 2
    s = hp * wp
    pads = wp + 1
    sp = _round_up(s + 2 * pads, 16)

    xt = jnp.transpose(x, (0, 2, 3, 4, 1)).astype(bf)
    xf = jnp.pad(xt, ((0, 0), (1, 1), (1, 1), (1, 1), (0, 0)))
    xf = xf.reshape(n, dp, s, c)
    xf = jnp.pad(xf, ((0, 0), (0, 0), (pads, sp - s - pads), (0, 0)))

    ii = jnp.arange(sp) - pads
    hh = ii // wp
    ww = ii % wp
    smask = ((ii >= 0) & (ii < s) & (hh >= 1) & (hh <= h)
             & (ww >= 1) & (ww <= w)).astype(jnp.float32).reshape(sp, 1)

    c0 = b0_w.shape[1]
    c1 = b1a_w.shape[1]
    c2 = b2a_w.shape[1]
    c1b = b1b_w.shape[-1]
    c2b = b2b_w.shape[-1]
    c3 = b3_w.shape[1]
    couts = c0 + c1b + c2b + c3

    db = d // 2 if d % 2 == 0 else d

    w0f = (b0_w * b0_s[None, :]).astype(bf)
    w12 = jnp.concatenate([b1a_w * b1a_s[None, :],
                           b2a_w * b2a_s[None, :]], axis=1).astype(bf)
    b12 = jnp.concatenate([b1a_b, b2a_b]).reshape(1, c1 + c2)
    w1f = (b1b_w * b1b_s).reshape(3, 9 * c1, c1b).astype(bf)
    w2f = (b2b_w * b2b_s).reshape(27 * c2, c2b).astype(bf)
    w3f = (b3_w * b3_s[None, :]).astype(bf)

    out = pl.pallas_call(
        functools.partial(_mixed_kernel, DB=db, D=d, H=h, W=w, C1=c1, C2=c2,
                          SP=sp, PADS=pads),
        out_shape=jax.ShapeDtypeStruct((n, d, h, w, couts), jnp.float32),
        grid_spec=pltpu.PrefetchScalarGridSpec(
            num_scalar_prefetch=0,
            grid=(n, d // db),
            in_specs=[
                pl.BlockSpec((pl.Squeezed(), dp, sp, c),
                             lambda ni, di: (ni, 0, 0, 0)),
                pl.BlockSpec((c, c0), lambda ni, di: (0, 0)),
                pl.BlockSpec((1, c0), lambda ni, di: (0, 0)),
                pl.BlockSpec((c, c1 + c2), lambda ni, di: (0, 0)),
                pl.BlockSpec((1, c1 + c2), lambda ni, di: (0, 0)),
                pl.BlockSpec((3, 9 * c1, c1b), lambda ni, di: (0, 0, 0)),
                pl.BlockSpec((1, c1b), lambda ni, di: (0, 0)),
                pl.BlockSpec((27 * c2, c2b), lambda ni, di: (0, 0)),
                pl.BlockSpec((1, c2b), lambda ni, di: (0, 0)),
                pl.BlockSpec((c, c3), lambda ni, di: (0, 0)),
                pl.BlockSpec((1, c3), lambda ni, di: (0, 0)),
                pl.BlockSpec((sp, 1), lambda ni, di: (0, 0)),
            ],
            out_specs=pl.BlockSpec((pl.Squeezed(), db, h, w, couts),
                                   lambda ni, di: (ni, di, 0, 0, 0)),
        ),
        compiler_params=pltpu.CompilerParams(
            dimension_semantics=("parallel", "parallel"),
            vmem_limit_bytes=60 * 1024 * 1024,
        ),
    )(xf, w0f, b0_b.reshape(1, c0), w12, b12,
      w1f, b1b_b.reshape(1, c1b), w2f, b2b_b.reshape(1, c2b),
      w3f, b3_b.reshape(1, c3), smask)
    return jnp.transpose(out, (0, 4, 1, 2, 3))
